# baseline (speedup 1.0000x reference)
import jax
import jax.numpy as jnp
from jax import lax
from jax.experimental import pallas as pl
from jax.experimental.pallas import tpu as pltpu
from jax.experimental.pallas import tpu_sc as plsc

_B, _C, _N, _K = 8, 64, 2048, 16
_KEEP = _N // 2
_RT = 256
_NT = _N // _RT
_TOPP = 4
_PAD = _KEEP + 1

_DN_T = (((0,), (0,)), ((), ()))
_DN_N = (((1,), (0,)), ((), ()))
_DN_RR = (((1,), (1,)), ((), ()))
_F32 = jnp.float32


def _dot(a, b, dn):
    return lax.dot_general(a, b, dn, preferred_element_type=_F32)


def _dotx(a, b, dn):
    return lax.dot_general(a, b, dn, precision=lax.Precision.HIGHEST,
                           preferred_element_type=_F32)


def _edge_pool_body(feat_ref, featT_ref, xx_ref, xxT_ref, wpack_ref,
                    v_ref, idx_ref, scol_ref, srow_ref):
    X = feat_ref[0]
    XT = featT_ref[0]
    xx_row = xx_ref[0]
    xx_colf = xxT_ref[0]
    w_row = wpack_ref[0:1, :]
    bias = wpack_ref[1:2, 0:1]
    bidx = pl.program_id(0)

    lane_iota = lax.broadcasted_iota(jnp.int32, (1, _N), 1)
    k_iota = lax.broadcasted_iota(jnp.int32, (1, _K), 1)
    I_rt = (lax.broadcasted_iota(jnp.int32, (_RT, _RT), 0) ==
            lax.broadcasted_iota(jnp.int32, (_RT, _RT), 1)).astype(_F32)
    I_2c = (lax.broadcasted_iota(jnp.int32, (2 * _C, 2 * _C), 0) ==
            lax.broadcasted_iota(jnp.int32, (2 * _C, 2 * _C), 1)).astype(_F32)
    neg_inf = _F32(-jnp.inf)

    w_col = _dotx(I_2c, w_row, _DN_RR)
    p_row = _dotx(w_row[:, 0:_C], X, _DN_N)

    for rt in range(_NT):
        sl = slice(rt * _RT, (rt + 1) * _RT)
        Xn = X[:, sl]
        G = _dot(Xn, X, _DN_T)
        inner = -2.0 * G
        xx_col = xx_colf[sl, :]
        D = ((-xx_col) - inner) - xx_row
        Dw = D
        jlist, plist = [], []
        for _ in range(_K):
            m = jnp.max(Dw, axis=1, keepdims=True)
            jidx = jnp.min(jnp.where(Dw == m, lane_iota, _N),
                           axis=1, keepdims=True)
            oh = (lane_iota == jidx)
            Dw = jnp.where(oh, neg_inf, Dw)
            plist.append(jnp.max(jnp.where(oh, p_row, neg_inf),
                                 axis=1, keepdims=True))
            jlist.append(jidx)
        Jmat = jnp.concatenate(jlist, axis=1)
        Pmat = jnp.concatenate(plist, axis=1)

        XnT = XT[sl, :]
        smax = jnp.full((_RT, 1), neg_inf, _F32)
        Pw = Pmat
        for _ in range(_TOPP):
            pm = jnp.max(Pw, axis=1, keepdims=True)
            kidx = jnp.min(jnp.where(Pw == pm, k_iota, _K),
                           axis=1, keepdims=True)
            ohk = (k_iota == kidx)
            Pw = jnp.where(ohk, neg_inf, Pw)
            jc = jnp.sum(jnp.where(ohk, Jmat, 0), axis=1, keepdims=True)
            ohc = (lane_iota == jc).astype(_F32)
            Xm = _dotx(ohc, XT, _DN_N)
            EF = jnp.concatenate([Xm - XnT, XnT], axis=1)
            s_t = _dot(EF, w_col, _DN_N) + bias
            smax = jnp.maximum(smax, s_t)
        scol_ref[sl, :] = smax
        srow_ref[0:1, sl] = _dotx(smax, I_rt, _DN_T)
        v_ref[0, sl, 0:_C] = XnT * jnp.tanh(smax)

    s_row = srow_ref[0:1, :]
    for rt in range(_NT):
        sl = slice(rt * _RT, (rt + 1) * _RT)
        s_col = scol_ref[sl, :]
        n_iota = lax.broadcasted_iota(jnp.int32, (_RT, 1), 0) + rt * _RT
        gt = (s_row > s_col) | ((s_row == s_col) & (lane_iota < n_iota))
        rank = jnp.sum(gt.astype(jnp.int32), axis=1, keepdims=True)
        rank = jnp.minimum(rank, _KEEP) + bidx * _PAD
        rank_row = _dotx(rank.astype(_F32), I_rt, _DN_T)
        idx_ref[0, 0:1, sl] = rank_row.astype(jnp.int32)


_SC_W = 32
_ROWS_PER_W = _B * _N // _SC_W
_CHUNK = 128
_NCHUNK = _ROWS_PER_W // _CHUNK


def _make_sc_scatter():
    mesh = plsc.VectorSubcoreMesh(core_axis_name="c", subcore_axis_name="s")

    def body(v_hbm, idx_hbm, out_hbm, idxv, rowsv, sem):
        wid = lax.axis_index("s") * 2 + lax.axis_index("c")
        base = wid * _ROWS_PER_W
        pltpu.sync_copy(idx_hbm.at[wid], idxv)
        pltpu.sync_copy(v_hbm.at[pl.ds(base, _ROWS_PER_W)], rowsv)
        for j in range(_NCHUNK):
            pltpu.async_copy(rowsv.at[pl.ds(j * _CHUNK, _CHUNK)],
                             out_hbm.at[idxv.at[j]], sem).wait()

    import functools
    return functools.partial(
        pl.kernel,
        mesh=mesh,
        out_type=jax.ShapeDtypeStruct((_B * _PAD, 128), jnp.float32),
        scratch_types=[
            pltpu.VMEM((_NCHUNK, _CHUNK), jnp.int32),
            pltpu.VMEM((_ROWS_PER_W, 128), jnp.float32),
            pltpu.SemaphoreType.DMA,
        ],
    )(body)


def kernel(feat, W, b):
    featT = jnp.transpose(feat, (0, 2, 1))
    xx = jnp.sum(feat * feat, axis=1)
    xx3 = xx.reshape(_B, 1, _N)
    xxT = xx.reshape(_B, _N, 1)
    wpack = jnp.zeros((8, 2 * _C), _F32)
    wpack = wpack.at[0, :].set(W[0, :, 0, 0])
    wpack = wpack.at[1, 0].set(b[0])
    V, idxg = pl.pallas_call(
        _edge_pool_body,
        grid=(_B,),
        in_specs=[
            pl.BlockSpec((1, _C, _N), lambda i: (i, 0, 0)),
            pl.BlockSpec((1, _N, _C), lambda i: (i, 0, 0)),
            pl.BlockSpec((1, 1, _N), lambda i: (i, 0, 0)),
            pl.BlockSpec((1, _N, 1), lambda i: (i, 0, 0)),
            pl.BlockSpec((8, 2 * _C), lambda i: (0, 0)),
        ],
        out_specs=[
            pl.BlockSpec((1, _N, 128), lambda i: (i, 0, 0)),
            pl.BlockSpec((1, 1, _N), lambda i: (i, 0, 0)),
        ],
        out_shape=[
            jax.ShapeDtypeStruct((_B, _N, 128), _F32),
            jax.ShapeDtypeStruct((_B, 1, _N), jnp.int32),
        ],
        scratch_shapes=[
            pltpu.VMEM((_N, 1), _F32),
            pltpu.VMEM((1, _N), _F32),
        ],
    )(feat, featT, xx3, xxT, wpack)
    vflat = V.reshape(_B * _N, 128)
    idx3 = idxg.reshape(_SC_W, _NCHUNK, _CHUNK)
    outpad = _make_sc_scatter()(vflat, idx3)
    out = outpad.reshape(_B, _PAD, 128)[:, :_KEEP, 0:_C]
    return jnp.transpose(out, (0, 2, 1))
